# baseline (speedup 1.0000x reference)
import jax
import jax.numpy as jnp
from jax import lax
from jax.experimental import pallas as pl
from jax.experimental.pallas import tpu as pltpu
from jax.experimental.pallas import tpu_sc as plsc

B, TX, S, RR, COLS = 16, 4, 2, 14, 4096
NSC = 3276
LANES = 16
NGRP = (NSC + LANES - 1) // LANES
IDX_PAD = NGRP * LANES
NUM_CORES = 2
NUM_SUBCORES = 16
NW = NUM_CORES * NUM_SUBCORES
NSLAB = B * TX * RR
G = NSLAB // 8
IN_W = S * COLS
JT = (NSC + 127) // 128
OUT_W = JT * S * 128


def _body(x_hbm, idx_hbm, out_hbm, idx_v, row_v, stage_v):
    wid = lax.axis_index("s") * NUM_CORES + lax.axis_index("c")
    lane = lax.iota(jnp.int32, LANES)

    pltpu.sync_copy(idx_hbm, idx_v)

    def chunk(cid):
        pltpu.sync_copy(x_hbm.at[cid], row_v)

        def grp(g):
            o = g * LANES
            iv = idx_v[pl.ds(o, LANES)]
            ivt = ((iv >> 7) << 8) + (iv & 127)
            ob = ((o >> 7) << 8) + (o & 127)
            for r in range(8):
                rs = jnp.full((LANES,), r, jnp.int32)
                for s in range(S):
                    vals = plsc.load_gather(
                        row_v, [rs, ivt + jnp.int32(s * 128)]
                    )
                    stage_v[r, pl.ds(ob + s * 128, LANES)] = vals

        plsc.parallel_loop(0, NGRP, 1, unroll=2)(grp)
        pltpu.sync_copy(stage_v, out_hbm.at[cid])

    for k in range(4):
        if k < 3:
            chunk(wid + NW * k)
        else:

            @pl.when(wid + NW * 3 < G)
            def _():
                chunk(wid + NW * 3)


_gather = pl.kernel(
    _body,
    out_type=jax.ShapeDtypeStruct((G, 8, OUT_W), jnp.float32),
    mesh=plsc.VectorSubcoreMesh(core_axis_name="c", subcore_axis_name="s"),
    scratch_types=[
        pltpu.VMEM((IDX_PAD,), jnp.int32),
        pltpu.VMEM((8, IN_W), jnp.float32),
        pltpu.VMEM((8, OUT_W), jnp.float32),
    ],
    compiler_params=pltpu.CompilerParams(needs_layout_passes=False),
)


@jax.jit
def kernel(inputs, sc_ind):
    x = (
        inputs.transpose(0, 1, 3, 2, 4)
        .reshape(B, TX, RR, S, COLS // 128, 128)
        .transpose(0, 1, 2, 4, 3, 5)
        .reshape(G, 8, IN_W)
    )
    idx = jnp.concatenate(
        [sc_ind.astype(jnp.int32), jnp.zeros((IDX_PAD - NSC,), jnp.int32)]
    )
    out = _gather(x, idx)
    return (
        out.reshape(B, TX, RR, JT, S, 128)
        .transpose(0, 1, 4, 2, 3, 5)
        .reshape(B, TX, S, RR, JT * 128)[..., :NSC]
    )


# speedup vs baseline: 2.6809x; 2.6809x over previous
import jax
import jax.numpy as jnp
from jax import lax
from jax.experimental import pallas as pl
from jax.experimental.pallas import tpu as pltpu
from jax.experimental.pallas import tpu_sc as plsc

B, TX, S, RR, COLS = 16, 4, 2, 14, 4096
NSC = 3276
LANES = 16
NGRP = (NSC + LANES - 1) // LANES
IDX_PAD = NGRP * LANES
TAIL = NSC - (NGRP - 1) * LANES
NUM_CORES = 2
NUM_SUBCORES = 16
NW = NUM_CORES * NUM_SUBCORES
UNITS = B * TX
IN_ROWS = UNITS * RR * 64
CHUNKS = tuple((2, r0) for r0 in range(0, RR, 2))


def _body(x_hbm, idx_hbm, out_hbm, idx_v, row_v, stage_v):
    wid = lax.axis_index("s") * NUM_CORES + lax.axis_index("c")
    lane = lax.iota(jnp.int32, LANES)

    pltpu.sync_copy(idx_hbm, idx_v)

    tail_o = (NGRP - 1) * LANES
    tail_mask = lane < TAIL
    tail_pos = jnp.minimum(jnp.int32(tail_o) + lane, NSC - 1)

    def unit(u):
        for n, r0 in CHUNKS:
            pltpu.sync_copy(
                x_hbm.at[pl.ds((u * RR + r0) * 64, n * 64)],
                row_v.at[pl.ds(0, n * 64)],
            )

            def grp(g, n=n, r0=r0):
                o = g * LANES
                iv = idx_v[pl.ds(o, LANES)]
                ivr = (iv >> 7) << 1
                col = iv & 127
                for rr in range(n):
                    for s in range(S):
                        vals = plsc.load_gather(
                            row_v, [ivr + jnp.int32(rr * 64 + s), col]
                        )
                        stage_v[s, r0 + rr, pl.ds(o, LANES)] = vals

            plsc.parallel_loop(0, NGRP - 1, 1, unroll=2)(grp)

            iv = idx_v[pl.ds(tail_o, LANES)]
            ivr = (iv >> 7) << 1
            col = iv & 127
            for rr in range(n):
                for s in range(S):
                    vals = plsc.load_gather(
                        row_v, [ivr + jnp.int32(rr * 64 + s), col]
                    )
                    plsc.store_scatter(
                        stage_v,
                        [
                            jnp.full((LANES,), s, jnp.int32),
                            jnp.full((LANES,), r0 + rr, jnp.int32),
                            tail_pos,
                        ],
                        vals,
                        mask=tail_mask,
                    )
        for s in range(S):
            pltpu.sync_copy(stage_v.at[s], out_hbm.at[u * S + s])

    unit(wid * 2)
    unit(wid * 2 + 1)


_gather = pl.kernel(
    _body,
    out_type=jax.ShapeDtypeStruct((UNITS * S, RR, NSC), jnp.float32),
    mesh=plsc.VectorSubcoreMesh(core_axis_name="c", subcore_axis_name="s"),
    scratch_types=[
        pltpu.VMEM((IDX_PAD,), jnp.int32),
        pltpu.VMEM((2 * 64, 128), jnp.float32),
        pltpu.VMEM((S, RR, NSC), jnp.float32),
    ],
    compiler_params=pltpu.CompilerParams(needs_layout_passes=False),
)


@jax.jit
def kernel(inputs, sc_ind):
    x = (
        inputs.transpose(0, 1, 3, 2, 4)
        .reshape(B, TX, RR, S, COLS // 128, 128)
        .transpose(0, 1, 2, 4, 3, 5)
        .reshape(IN_ROWS, 128)
    )
    idx = jnp.concatenate(
        [sc_ind.astype(jnp.int32), jnp.zeros((IDX_PAD - NSC,), jnp.int32)]
    )
    out = _gather(x, idx)
    return out.reshape(B, TX, S, RR, NSC)


# speedup vs baseline: 2.7984x; 1.0439x over previous
import jax
import jax.numpy as jnp
from jax import lax
from jax.experimental import pallas as pl
from jax.experimental.pallas import tpu as pltpu
from jax.experimental.pallas import tpu_sc as plsc

B, TX, S, RR, COLS = 16, 4, 2, 14, 4096
NSC = 3276
LANES = 16
NGRP = (NSC + LANES - 1) // LANES
IDX_PAD = NGRP * LANES
TAIL = NSC - (NGRP - 1) * LANES
NUM_CORES = 2
NUM_SUBCORES = 16
NW = NUM_CORES * NUM_SUBCORES
UNITS = B * TX
IN_ROWS = UNITS * RR * 64


def _body(x_hbm, idx_hbm, out_hbm, idx_v, row_a, row_b, stage_v, sems):
    wid = lax.axis_index("s") * NUM_CORES + lax.axis_index("c")
    lane = lax.iota(jnp.int32, LANES)

    pltpu.sync_copy(idx_hbm, idx_v)

    tail_o = (NGRP - 1) * LANES
    tail_mask = lane < TAIL
    tail_pos = jnp.minimum(jnp.int32(tail_o) + lane, NSC - 1)
    bufs = (row_a, row_b)

    def in_dma(u, r):
        return pltpu.async_copy(
            x_hbm.at[pl.ds((u * RR + r) * 64, 64)], bufs[r % 2], sems.at[r % 2]
        )

    def unit(u, drain):
        pend = in_dma(u, 0)
        for d in drain:
            d.wait()
        for r in range(RR):
            nxt = in_dma(u, r + 1) if r + 1 < RR else None
            pend.wait()
            pend = nxt
            row_v = bufs[r % 2]

            def grp(g, r=r, row_v=row_v):
                o = g * LANES
                iv = idx_v[pl.ds(o, LANES)]
                ivr = (iv >> 7) << 1
                col = iv & 127
                for s in range(S):
                    vals = plsc.load_gather(row_v, [ivr + jnp.int32(s), col])
                    stage_v[s, r, pl.ds(o, LANES)] = vals

            plsc.parallel_loop(0, NGRP - 1, 1, unroll=4)(grp)

            iv = idx_v[pl.ds(tail_o, LANES)]
            ivr = (iv >> 7) << 1
            col = iv & 127
            for s in range(S):
                vals = plsc.load_gather(row_v, [ivr + jnp.int32(s), col])
                plsc.store_scatter(
                    stage_v,
                    [
                        jnp.full((LANES,), s, jnp.int32),
                        jnp.full((LANES,), r, jnp.int32),
                        tail_pos,
                    ],
                    vals,
                    mask=tail_mask,
                )
        return [
            pltpu.async_copy(
                stage_v.at[s], out_hbm.at[u * S + s], sems.at[2 + s]
            )
            for s in range(S)
        ]

    drain = unit(wid * 2, [])
    for d in unit(wid * 2 + 1, drain):
        d.wait()


_gather = pl.kernel(
    _body,
    out_type=jax.ShapeDtypeStruct((UNITS * S, RR, NSC), jnp.float32),
    mesh=plsc.VectorSubcoreMesh(core_axis_name="c", subcore_axis_name="s"),
    scratch_types=[
        pltpu.VMEM((IDX_PAD,), jnp.int32),
        pltpu.VMEM((64, 128), jnp.float32),
        pltpu.VMEM((64, 128), jnp.float32),
        pltpu.VMEM((S, RR, NSC), jnp.float32),
        pltpu.SemaphoreType.DMA((4,)),
    ],
    compiler_params=pltpu.CompilerParams(needs_layout_passes=False),
)


@jax.jit
def kernel(inputs, sc_ind):
    x = (
        inputs.transpose(0, 1, 3, 2, 4)
        .reshape(B, TX, RR, S, COLS // 128, 128)
        .transpose(0, 1, 2, 4, 3, 5)
        .reshape(IN_ROWS, 128)
    )
    idx = jnp.concatenate(
        [sc_ind.astype(jnp.int32), jnp.zeros((IDX_PAD - NSC,), jnp.int32)]
    )
    out = _gather(x, idx)
    return out.reshape(B, TX, S, RR, NSC)
